# Initial kernel scaffold; baseline (speedup 1.0000x reference)
import jax
import jax.numpy as jnp
from jax import lax
from jax.experimental import pallas as pl
from jax.experimental.pallas import tpu as pltpu

_B, _S, _F = 16, 2048, 128
_RB = 256
_NSTEP = _S // _RB
_BB = _B // _NSTEP
_BIG = float(jnp.finfo(jnp.float32).max)


def _shift_lanes(v, k, fill):
    return jnp.concatenate(
        [jnp.full((1, k), fill, v.dtype), v[:, :_S - k]], axis=1)


def _cumsum_row(v):
    k = 1
    while k < _S:
        v = v + _shift_lanes(v, k, 0.0)
        k *= 2
    return v


def _cummin_row(v):
    k = 1
    while k < _S:
        v = jnp.minimum(v, _shift_lanes(v, k, _BIG))
        k *= 2
    return v


def _row_update(prev, drow):
    shifted = _shift_lanes(prev, 1, _BIG)
    m = jnp.minimum(prev, shifted)
    b = drow + m
    c = _cumsum_row(drow)
    return c + _cummin_row(b - c)


def _fused_kernel(pred_ref, target_ref, x_ref, y_ref, out_ref,
                  d_scr, dprev_scr, acc_ref):
    i = pl.program_id(0)

    part = jnp.sum(jnp.abs(pred_ref[...] - target_ref[...]))

    @pl.when(i == 0)
    def _():
        acc_ref[0] = 0.0

    acc_ref[0] = acc_ref[0] + part

    xb = x_ref[...]
    y = y_ref[...]
    xsq = jnp.sum(xb * xb, axis=1, keepdims=True)
    ysq = jnp.sum(y * y, axis=1, keepdims=True)
    lhs = jnp.concatenate(
        [-2.0 * xb, xsq, jnp.ones((_RB, 1), jnp.float32)], axis=1)
    rhs = jnp.concatenate(
        [y, jnp.ones((_S, 1), jnp.float32), ysq], axis=1)
    sq = lax.dot_general(lhs, rhs, (((1,), (1,)), ((), ())),
                         preferred_element_type=jnp.float32)
    d_scr[...] = jnp.sqrt(jnp.maximum(sq, 1e-12))

    @pl.when(i == 0)
    def _():
        d8 = d_scr[0:8, :]
        row = _cumsum_row(d8[0:1, :])
        for r8 in range(1, 8):
            row = _row_update(row, d8[r8:r8 + 1, :])
        dprev_scr[...] = row

    start = jnp.where(i == 0, 1, 0)

    def outer(rt, carry):
        base = pl.multiple_of(rt * 8, 8)
        d8 = d_scr[pl.ds(base, 8), :]
        for r8 in range(8):
            carry = _row_update(carry, d8[r8:r8 + 1, :])
        return carry

    final = lax.fori_loop(start, _RB // 8, outer, dprev_scr[...])
    dprev_scr[...] = final

    @pl.when(i == _NSTEP - 1)
    def _():
        mae = acc_ref[0] / float(_B * _S * _F)
        dtw = final[0, _S - 1] / float(_S * _F)
        out_ref[0, 0] = 0.5 * mae + 0.5 * dtw


def kernel(pred, target):
    x = pred[0]
    y = target[0]
    out = pl.pallas_call(
        _fused_kernel,
        grid=(_NSTEP,),
        in_specs=[
            pl.BlockSpec((_BB, _S, _F), lambda i: (i, 0, 0)),
            pl.BlockSpec((_BB, _S, _F), lambda i: (i, 0, 0)),
            pl.BlockSpec((_RB, _F), lambda i: (i, 0)),
            pl.BlockSpec((_S, _F), lambda i: (0, 0)),
        ],
        out_specs=pl.BlockSpec((1, 1), lambda i: (0, 0)),
        out_shape=jax.ShapeDtypeStruct((1, 1), jnp.float32),
        scratch_shapes=[
            pltpu.VMEM((_RB, _S), jnp.float32),
            pltpu.VMEM((1, _S), jnp.float32),
            pltpu.SMEM((1,), jnp.float32),
        ],
        compiler_params=pltpu.CompilerParams(
            dimension_semantics=("arbitrary",),
        ),
    )(pred, target, x, y)
    return out[0, 0]


# speedup vs baseline: 3.2029x; 3.2029x over previous
import jax
import jax.numpy as jnp
from jax import lax
from jax.experimental import pallas as pl
from jax.experimental.pallas import tpu as pltpu

_B, _S, _F = 16, 2048, 128
_RB = 256
_NSTEP = _S // _RB
_BB = _B // _NSTEP
_BIG = float(jnp.finfo(jnp.float32).max)


def _shift_lanes(v, k, fill):
    return jnp.concatenate(
        [jnp.full((1, k), fill, v.dtype), v[:, :_S - k]], axis=1)


def _cumsum_row(v):
    k = 1
    while k < _S:
        v = v + _shift_lanes(v, k, 0.0)
        k *= 2
    return v


def _cummin_row(v):
    k = 1
    while k < _S:
        v = jnp.minimum(v, _shift_lanes(v, k, _BIG))
        k *= 2
    return v


def _row_update(prev, drow):
    shifted = _shift_lanes(prev, 1, _BIG)
    m = jnp.minimum(prev, shifted)
    b = drow + m
    c = _cumsum_row(drow)
    return c + _cummin_row(b - c)


def _fused_kernel(pred_ref, target_ref, x_ref, y_ref, out_ref,
                  d_scr, dprev_scr, acc_ref):
    i = pl.program_id(0)

    part = jnp.sum(jnp.abs(pred_ref[...] - target_ref[...]))

    @pl.when(i == 0)
    def _():
        acc_ref[0] = 0.0

    acc_ref[0] = acc_ref[0] + part

    xb = x_ref[...]
    y = y_ref[...]
    xsq = jnp.sum(xb * xb, axis=1, keepdims=True)
    ysq = jnp.sum(y * y, axis=1, keepdims=True)
    lhs = jnp.concatenate(
        [-2.0 * xb, xsq, jnp.ones((_RB, 1), jnp.float32)], axis=1)
    rhs = jnp.concatenate(
        [y, jnp.ones((_S, 1), jnp.float32), ysq], axis=1)
    sq = lax.dot_general(lhs, rhs, (((1,), (1,)), ((), ())),
                         preferred_element_type=jnp.float32)
    d_scr[...] = jnp.sqrt(jnp.maximum(sq, 1e-12))

    @pl.when(i == 0)
    def _():
        d8 = d_scr[0:8, :]
        row = _cumsum_row(d8[0:1, :])
        for r8 in range(1, 8):
            row = _row_update(row, d8[r8:r8 + 1, :])
        dprev_scr[...] = row

    start = jnp.where(i == 0, 1, 0)

    def outer(rt, carry):
        base = pl.multiple_of(rt * 8, 8)
        d8 = d_scr[pl.ds(base, 8), :]
        for r8 in range(8):
            carry = _row_update(carry, d8[r8:r8 + 1, :])
        return carry

    final = lax.fori_loop(start, _RB // 8, outer, dprev_scr[...])
    dprev_scr[...] = final

    @pl.when(i == _NSTEP - 1)
    def _():
        mae = acc_ref[0] / float(_B * _S * _F)
        dtw = final[0, _S - 1] / float(_S * _F)
        out_ref[...] = (0.5 * mae + 0.5 * dtw) * jnp.ones((1, 1), jnp.float32)


def kernel(pred, target):
    x = pred[0]
    y = target[0]
    out = pl.pallas_call(
        _fused_kernel,
        grid=(_NSTEP,),
        in_specs=[
            pl.BlockSpec((_BB, _S, _F), lambda i: (i, 0, 0)),
            pl.BlockSpec((_BB, _S, _F), lambda i: (i, 0, 0)),
            pl.BlockSpec((_RB, _F), lambda i: (i, 0)),
            pl.BlockSpec((_S, _F), lambda i: (0, 0)),
        ],
        out_specs=pl.BlockSpec((1, 1), lambda i: (0, 0)),
        out_shape=jax.ShapeDtypeStruct((1, 1), jnp.float32),
        scratch_shapes=[
            pltpu.VMEM((_RB, _S), jnp.float32),
            pltpu.VMEM((1, _S), jnp.float32),
            pltpu.SMEM((1,), jnp.float32),
        ],
        compiler_params=pltpu.CompilerParams(
            dimension_semantics=("arbitrary",),
        ),
    )(pred, target, x, y)
    return out[0, 0]
